# Initial kernel scaffold; baseline (speedup 1.0000x reference)
import jax
import jax.numpy as jnp
from jax.experimental import pallas as pl


def kernel(input_ids, table):
    raise NotImplementedError("write your pallas kernel here")


# speedup vs baseline: 1.8743x; 1.8743x over previous
import functools

import jax
import jax.numpy as jnp
from jax import lax
from jax.experimental import pallas as pl
from jax.experimental.pallas import tpu as pltpu
from jax.experimental.pallas import tpu_sc as plsc

BATCH = 16384
HIST = 50
DIM = 64

NC = 2
NS = 16
NW = NC * NS

CHUNK = 128
TOTAL = BATCH * HIST
N_CHUNKS = TOTAL // CHUNK
CPW = N_CHUNKS // NW
G = 4
NBUF = 2
GPW = CPW // G


def _cbow_body(idx_hbm, table_hbm, out_hbm, idx_v, rows_v, sem0, sem1):
    sems = (sem0, sem1)
    wid = lax.axis_index("s") * NC + lax.axis_index("c")
    chunk0 = wid * CPW

    pltpu.sync_copy(idx_hbm.at[pl.ds(chunk0, CPW)], idx_v)

    def fire(g, b):
        for j in range(G):
            c = g * G + j
            pltpu.async_copy(table_hbm.at[idx_v.at[c]], rows_v.at[b, j], sems[b])

    def drain(b):
        for j in range(G):
            pltpu.make_async_copy(table_hbm.at[pl.ds(0, CHUNK)],
                                  rows_v.at[b, j], sems[b]).wait()

    def store(g, b):
        pltpu.sync_copy(rows_v.at[b], out_hbm.at[pl.ds(chunk0 + g * G, G)])

    for b in range(NBUF):
        fire(b, b)

    def step(i, _):
        for b in range(NBUF):
            g = i * NBUF + b
            drain(b)
            store(g, b)
            fire(g + NBUF, b)
        return _

    lax.fori_loop(0, GPW // NBUF - 1, step, None, unroll=False)

    for b in range(NBUF):
        g = GPW - NBUF + b
        drain(b)
        store(g, b)


@functools.partial(jax.jit, static_argnames=())
def kernel(input_ids, table):
    idx = input_ids.reshape(N_CHUNKS, CHUNK).astype(jnp.int32)
    mesh = plsc.VectorSubcoreMesh(core_axis_name="c", subcore_axis_name="s",
                                  num_cores=NC, num_subcores=NS)
    out = pl.kernel(
        _cbow_body,
        out_type=jax.ShapeDtypeStruct((N_CHUNKS, CHUNK, DIM), jnp.float32),
        mesh=mesh,
        scratch_types=[
            pltpu.VMEM((CPW, CHUNK), jnp.int32),
            pltpu.VMEM((NBUF, G, CHUNK, DIM), jnp.float32),
            pltpu.SemaphoreType.DMA,
            pltpu.SemaphoreType.DMA,
        ],
        compiler_params=pltpu.CompilerParams(use_tc_tiling_on_sc=False),
    )(idx, table)
    return out.reshape(BATCH, HIST, DIM)
